# baseline (speedup 1.0000x reference)
import functools

import jax
import jax.numpy as jnp
from jax import lax
from jax.experimental import pallas as pl
from jax.experimental.pallas import tpu as pltpu
from jax.experimental.pallas import tpu_sc as plsc

_C, _H, _W = 3, 32, 32
_OC, _OH, _OW = 16, 16, 16
_KH, _KW = 4, 4
_PREV = _C * _H * _W
_OUT = _OC * _OH * _OW
_ROWS_A = _PREV + 1
_COLS_A = _OUT + 1
_D_IN = 1025

_A_BLK_R, _A_BLK_C = 1024, 256
_N_TILE = 1024


def _a_build_kernel(w_ref, out_ref):
    i = pl.program_id(0)
    j = pl.program_id(1)

    def tap_block():
        iw2 = jax.lax.broadcasted_iota(jnp.int32, (_W, _OW), 0)
        ow2 = jax.lax.broadcasted_iota(jnp.int32, (_W, _OW), 1)
        kwv = iw2 - 2 * ow2 + 1
        tiles = []
        for kh in range(_KH):
            t = jnp.zeros((_W, _OW), jnp.float32)
            for kw in range(_KW):
                t = jnp.where(kwv == kw, w_ref[0, 0, kh, kw], t)
            tiles.append(t)
        zt = jnp.zeros((_W, _OW), jnp.float32)
        bands = []
        for ihv in range(_H):
            pieces = [zt] * _OH
            for kh in range(_KH):
                t2 = ihv + 1 - kh
                if t2 >= 0 and t2 % 2 == 0 and t2 // 2 < _OH:
                    pieces[t2 // 2] = tiles[kh]
            bands.append(jnp.concatenate(pieces, axis=1))
        return jnp.concatenate(bands, axis=0)

    def edge_block():
        return jnp.zeros((_A_BLK_R, _A_BLK_C), jnp.float32)

    out_ref[...] = jax.lax.cond((i < _C) & (j < _OC), tap_block, edge_block)


def _build_a(conv_weight, conv_bias):
    grid = (pl.cdiv(_ROWS_A, _A_BLK_R), pl.cdiv(_COLS_A, _A_BLK_C))
    a = pl.pallas_call(
        _a_build_kernel,
        grid=grid,
        in_specs=[
            pl.BlockSpec(
                (1, 1, _KH, _KW),
                lambda i, j: (jnp.minimum(j, _OC - 1), jnp.minimum(i, _C - 1), 0, 0),
            ),
        ],
        out_specs=pl.BlockSpec((_A_BLK_R, _A_BLK_C), lambda i, j: (i, j)),
        out_shape=jax.ShapeDtypeStruct((_ROWS_A, _COLS_A), jnp.float32),
    )(conv_weight)
    bias_row = jnp.concatenate(
        [jnp.repeat(conv_bias, _OUT // _OC), jnp.ones((1,), jnp.float32)]
    )
    return a.at[_PREV, :].set(bias_row)


_NW = 32
_RPW = _PREV // _NW
_RPC = 8
_CPW = _RPW // _RPC
_CHUNK = _RPC * _COLS_A
_BUF_N = 32784


def _a_sc_body(w_hbm, a_hbm, w_v, buf0, buf1, sem0, sem1):
    wid = lax.axis_index("s") * 2 + lax.axis_index("c")
    pltpu.sync_copy(w_hbm, w_v)
    bufs = (buf0, buf1)
    sems = (sem0, sem1)
    lanes = lax.iota(jnp.int32, 16)
    oc_col = lanes * (_OH * _OW)
    w_base = lanes * (_C * _KH * _KW)
    zeros16 = jnp.zeros((16,), jnp.float32)

    for b in range(2):
        for t in range(_BUF_N // 16):
            bufs[b][pl.ds(t * 16, 16)] = zeros16

    row0 = wid * _RPW

    def scatter_taps(buf, p_base, zero_mode):
        for rr in range(_RPC):
            p = p_base + rr
            cch = p // (_H * _W)
            rem = p % (_H * _W)
            ih = rem // _W
            iw = rem % _W
            rbase = rr * _COLS_A
            for kh in range(_KH):
                th = ih + 1 - kh
                oh = jnp.clip(th // 2, 0, _OH - 1)
                vh = (th >= 0) & (th < _H) & (th % 2 == 0)
                for kw in range(_KW):
                    tw = iw + 1 - kw
                    ow = jnp.clip(tw // 2, 0, _OW - 1)
                    vw = (tw >= 0) & (tw < _W) & (tw % 2 == 0)
                    mask = jnp.broadcast_to(vh & vw, (16,))
                    idx16 = oc_col + (rbase + oh * _OW + ow)
                    if zero_mode:
                        vals = zeros16
                    else:
                        vals = plsc.load_gather(
                            w_v, [w_base + cch * (_KH * _KW) + kh * _KW + kw])
                    plsc.store_scatter(buf, [idx16], vals, mask=mask)

    def loop_body(k2, carry):
        k = k2 * 2
        for b in range(2):
            kk = k + b

            @pl.when(kk >= 2)
            def _():
                pltpu.make_async_copy(
                    bufs[b].at[pl.ds(0, _CHUNK)],
                    a_hbm.at[pl.ds(0, _CHUNK)], sems[b]).wait()
                scatter_taps(bufs[b], row0 + (kk - 2) * _RPC, True)

            scatter_taps(bufs[b], row0 + kk * _RPC, False)
            pltpu.async_copy(
                bufs[b].at[pl.ds(0, _CHUNK)],
                a_hbm.at[pl.ds((row0 + kk * _RPC) * _COLS_A, _CHUNK)],
                sems[b])
        return carry

    lax.fori_loop(0, _CPW // 2, loop_body, 0)
    for b in range(2):
        pltpu.make_async_copy(
            bufs[b].at[pl.ds(0, _CHUNK)],
            a_hbm.at[pl.ds(0, _CHUNK)], sems[b]).wait()


def _build_a_sc(conv_weight, conv_bias):
    mesh = plsc.VectorSubcoreMesh(core_axis_name="c", subcore_axis_name="s")
    fn = functools.partial(
        pl.kernel,
        mesh=mesh,
        compiler_params=pltpu.CompilerParams(
            use_tc_tiling_on_sc=False, needs_layout_passes=False),
        out_type=jax.ShapeDtypeStruct((_ROWS_A * _COLS_A,), jnp.float32),
        scratch_types=[
            pltpu.VMEM((_OC * _C * _KH * _KW,), jnp.float32),
            pltpu.VMEM((_BUF_N,), jnp.float32),
            pltpu.VMEM((_BUF_N,), jnp.float32),
            pltpu.SemaphoreType.DMA,
            pltpu.SemaphoreType.DMA,
        ],
    )(_a_sc_body)
    a = fn(conv_weight.reshape(-1)).reshape(_ROWS_A, _COLS_A)
    bias_row = jnp.concatenate(
        [jnp.repeat(conv_bias, _OUT // _OC), jnp.ones((1,), jnp.float32)]
    )
    return a.at[_PREV, :].set(bias_row)


def _bounds_kernel(m_ref, a_ref, ce_ref, re_ref, low_ref, up_ref):
    b = jnp.dot(m_ref[...], a_ref[...], preferred_element_type=jnp.float32)
    t1 = jnp.dot(ce_ref[...], b, preferred_element_type=jnp.float32)
    t2 = jnp.dot(re_ref[...], jnp.abs(b), preferred_element_type=jnp.float32)
    low_ref[...] = t1 - t2
    up_ref[...] = t1 + t2


def _bounds(m, a, ce, re):
    n_tiles = pl.cdiv(_COLS_A, _N_TILE)
    low, up = pl.pallas_call(
        _bounds_kernel,
        grid=(n_tiles,),
        in_specs=[
            pl.BlockSpec((_D_IN, _ROWS_A), lambda n: (0, 0)),
            pl.BlockSpec((_ROWS_A, _N_TILE), lambda n: (0, n)),
            pl.BlockSpec((1, _D_IN), lambda n: (0, 0)),
            pl.BlockSpec((1, _D_IN), lambda n: (0, 0)),
        ],
        out_specs=[
            pl.BlockSpec((1, _N_TILE), lambda n: (0, n)),
            pl.BlockSpec((1, _N_TILE), lambda n: (0, n)),
        ],
        out_shape=[
            jax.ShapeDtypeStruct((1, n_tiles * _N_TILE), jnp.float32),
            jax.ShapeDtypeStruct((1, n_tiles * _N_TILE), jnp.float32),
        ],
    )(m, a, ce, re)
    return low, up


@jax.jit
def kernel(concrete_lower, concrete_upper, abstract_lower, abstract_upper,
           conv_weight, conv_bias, M, box_lower, box_upper):
    a = _build_a(conv_weight, conv_bias)
    c = (box_lower + box_upper) * 0.5
    r = (box_upper - box_lower) * 0.5
    ce = jnp.concatenate([c, jnp.ones((1,), jnp.float32)])[None, :]
    re = jnp.concatenate([r, jnp.zeros((1,), jnp.float32)])[None, :]
    low, up = _bounds(M, a, ce, re)
    out_dim = (_OC, _OH, _OW)
    lower_out = low[0, :_OUT].reshape(out_dim)
    upper_out = up[0, :_OUT].reshape(out_dim)
    return (lower_out, upper_out, a, a)


# speedup vs baseline: 1.0113x; 1.0113x over previous
import functools

import jax
import jax.numpy as jnp
from jax import lax
from jax.experimental import pallas as pl
from jax.experimental.pallas import tpu as pltpu
from jax.experimental.pallas import tpu_sc as plsc

_C, _H, _W = 3, 32, 32
_OC, _OH, _OW = 16, 16, 16
_KH, _KW = 4, 4
_PREV = _C * _H * _W
_OUT = _OC * _OH * _OW
_ROWS_A = _PREV + 1
_COLS_A = _OUT + 1
_D_IN = 1025

_A_BLK_R, _A_BLK_C = 1024, 256
_N_TILE = 256


def _a_build_kernel(w_ref, out_ref):
    i = pl.program_id(0)
    j = pl.program_id(1)

    def tap_block():
        iw2 = jax.lax.broadcasted_iota(jnp.int32, (_W, _OW), 0)
        ow2 = jax.lax.broadcasted_iota(jnp.int32, (_W, _OW), 1)
        kwv = iw2 - 2 * ow2 + 1
        tiles = []
        for kh in range(_KH):
            t = jnp.zeros((_W, _OW), jnp.float32)
            for kw in range(_KW):
                t = jnp.where(kwv == kw, w_ref[0, 0, kh, kw], t)
            tiles.append(t)
        zt = jnp.zeros((_W, _OW), jnp.float32)
        bands = []
        for ihv in range(_H):
            pieces = [zt] * _OH
            for kh in range(_KH):
                t2 = ihv + 1 - kh
                if t2 >= 0 and t2 % 2 == 0 and t2 // 2 < _OH:
                    pieces[t2 // 2] = tiles[kh]
            bands.append(jnp.concatenate(pieces, axis=1))
        return jnp.concatenate(bands, axis=0)

    def edge_block():
        return jnp.zeros((_A_BLK_R, _A_BLK_C), jnp.float32)

    out_ref[...] = jax.lax.cond((i < _C) & (j < _OC), tap_block, edge_block)


def _build_a(conv_weight, conv_bias):
    grid = (pl.cdiv(_ROWS_A, _A_BLK_R), pl.cdiv(_COLS_A, _A_BLK_C))
    a = pl.pallas_call(
        _a_build_kernel,
        grid=grid,
        in_specs=[
            pl.BlockSpec(
                (1, 1, _KH, _KW),
                lambda i, j: (jnp.minimum(j, _OC - 1), jnp.minimum(i, _C - 1), 0, 0),
            ),
        ],
        out_specs=pl.BlockSpec((_A_BLK_R, _A_BLK_C), lambda i, j: (i, j)),
        out_shape=jax.ShapeDtypeStruct((_ROWS_A, _COLS_A), jnp.float32),
    )(conv_weight)
    bias_row = jnp.concatenate(
        [jnp.repeat(conv_bias, _OUT // _OC), jnp.ones((1,), jnp.float32)]
    )
    return a.at[_PREV, :].set(bias_row)


_NW = 32
_RPW = _PREV // _NW
_RPC = 8
_CPW = _RPW // _RPC
_CHUNK = _RPC * _COLS_A
_BUF_N = 32784


def _a_sc_body(w_hbm, a_hbm, w_v, buf0, buf1, sem0, sem1):
    wid = lax.axis_index("s") * 2 + lax.axis_index("c")
    pltpu.sync_copy(w_hbm, w_v)
    bufs = (buf0, buf1)
    sems = (sem0, sem1)
    lanes = lax.iota(jnp.int32, 16)
    oc_col = lanes * (_OH * _OW)
    w_base = lanes * (_C * _KH * _KW)
    zeros16 = jnp.zeros((16,), jnp.float32)

    for b in range(2):
        for t in range(_BUF_N // 16):
            bufs[b][pl.ds(t * 16, 16)] = zeros16

    row0 = wid * _RPW

    def scatter_taps(buf, p_base, zero_mode):
        for rr in range(_RPC):
            p = p_base + rr
            cch = p // (_H * _W)
            rem = p % (_H * _W)
            ih = rem // _W
            iw = rem % _W
            rbase = rr * _COLS_A
            for kh in range(_KH):
                th = ih + 1 - kh
                oh = jnp.clip(th // 2, 0, _OH - 1)
                vh = (th >= 0) & (th < _H) & (th % 2 == 0)
                for kw in range(_KW):
                    tw = iw + 1 - kw
                    ow = jnp.clip(tw // 2, 0, _OW - 1)
                    vw = (tw >= 0) & (tw < _W) & (tw % 2 == 0)
                    mask = jnp.broadcast_to(vh & vw, (16,))
                    idx16 = oc_col + (rbase + oh * _OW + ow)
                    if zero_mode:
                        vals = zeros16
                    else:
                        vals = plsc.load_gather(
                            w_v, [w_base + cch * (_KH * _KW) + kh * _KW + kw])
                    plsc.store_scatter(buf, [idx16], vals, mask=mask)

    def loop_body(k2, carry):
        k = k2 * 2
        for b in range(2):
            kk = k + b

            @pl.when(kk >= 2)
            def _():
                pltpu.make_async_copy(
                    bufs[b].at[pl.ds(0, _CHUNK)],
                    a_hbm.at[pl.ds(0, _CHUNK)], sems[b]).wait()
                scatter_taps(bufs[b], row0 + (kk - 2) * _RPC, True)

            scatter_taps(bufs[b], row0 + kk * _RPC, False)
            pltpu.async_copy(
                bufs[b].at[pl.ds(0, _CHUNK)],
                a_hbm.at[pl.ds((row0 + kk * _RPC) * _COLS_A, _CHUNK)],
                sems[b])
        return carry

    lax.fori_loop(0, _CPW // 2, loop_body, 0)
    for b in range(2):
        pltpu.make_async_copy(
            bufs[b].at[pl.ds(0, _CHUNK)],
            a_hbm.at[pl.ds(0, _CHUNK)], sems[b]).wait()


def _build_a_sc(conv_weight, conv_bias):
    mesh = plsc.VectorSubcoreMesh(core_axis_name="c", subcore_axis_name="s")
    fn = functools.partial(
        pl.kernel,
        mesh=mesh,
        compiler_params=pltpu.CompilerParams(
            use_tc_tiling_on_sc=False, needs_layout_passes=False),
        out_type=jax.ShapeDtypeStruct((_ROWS_A * _COLS_A,), jnp.float32),
        scratch_types=[
            pltpu.VMEM((_OC * _C * _KH * _KW,), jnp.float32),
            pltpu.VMEM((_BUF_N,), jnp.float32),
            pltpu.VMEM((_BUF_N,), jnp.float32),
            pltpu.SemaphoreType.DMA,
            pltpu.SemaphoreType.DMA,
        ],
    )(_a_sc_body)
    a = fn(conv_weight.reshape(-1)).reshape(_ROWS_A, _COLS_A)
    bias_row = jnp.concatenate(
        [jnp.repeat(conv_bias, _OUT // _OC), jnp.ones((1,), jnp.float32)]
    )
    return a.at[_PREV, :].set(bias_row)


def _bounds_kernel(m_ref, a_ref, ce_ref, re_ref, low_ref, up_ref):
    b = jnp.dot(m_ref[...], a_ref[...], preferred_element_type=jnp.float32)
    t1 = jnp.dot(ce_ref[...], b, preferred_element_type=jnp.float32)
    t2 = jnp.dot(re_ref[...], jnp.abs(b), preferred_element_type=jnp.float32)
    low_ref[...] = t1 - t2
    up_ref[...] = t1 + t2


def _bounds(m, a, ce, re):
    n_tiles = pl.cdiv(_COLS_A, _N_TILE)
    low, up = pl.pallas_call(
        _bounds_kernel,
        grid=(n_tiles,),
        in_specs=[
            pl.BlockSpec((_D_IN, _ROWS_A), lambda n: (0, 0)),
            pl.BlockSpec((_ROWS_A, _N_TILE), lambda n: (0, n)),
            pl.BlockSpec((1, _D_IN), lambda n: (0, 0)),
            pl.BlockSpec((1, _D_IN), lambda n: (0, 0)),
        ],
        out_specs=[
            pl.BlockSpec((1, _N_TILE), lambda n: (0, n)),
            pl.BlockSpec((1, _N_TILE), lambda n: (0, n)),
        ],
        out_shape=[
            jax.ShapeDtypeStruct((1, n_tiles * _N_TILE), jnp.float32),
            jax.ShapeDtypeStruct((1, n_tiles * _N_TILE), jnp.float32),
        ],
    )(m, a, ce, re)
    return low, up


@jax.jit
def kernel(concrete_lower, concrete_upper, abstract_lower, abstract_upper,
           conv_weight, conv_bias, M, box_lower, box_upper):
    a = _build_a(conv_weight, conv_bias)
    c = (box_lower + box_upper) * 0.5
    r = (box_upper - box_lower) * 0.5
    ce = jnp.concatenate([c, jnp.ones((1,), jnp.float32)])[None, :]
    re = jnp.concatenate([r, jnp.zeros((1,), jnp.float32)])[None, :]
    low, up = _bounds(M, a, ce, re)
    out_dim = (_OC, _OH, _OW)
    lower_out = low[0, :_OUT].reshape(out_dim)
    upper_out = up[0, :_OUT].reshape(out_dim)
    return (lower_out, upper_out, a, a)


# speedup vs baseline: 1.0337x; 1.0222x over previous
import functools

import jax
import jax.numpy as jnp
from jax import lax
from jax.experimental import pallas as pl
from jax.experimental.pallas import tpu as pltpu
from jax.experimental.pallas import tpu_sc as plsc

_C, _H, _W = 3, 32, 32
_OC, _OH, _OW = 16, 16, 16
_KH, _KW = 4, 4
_PREV = _C * _H * _W
_OUT = _OC * _OH * _OW
_ROWS_A = _PREV + 1
_COLS_A = _OUT + 1
_D_IN = 1025

_A_BLK_R, _A_BLK_C = 1024, 256
_N_TILE = 512


def _a_build_kernel(w_ref, out_ref):
    i = pl.program_id(0)
    j = pl.program_id(1)

    def tap_block():
        iw2 = jax.lax.broadcasted_iota(jnp.int32, (_W, _OW), 0)
        ow2 = jax.lax.broadcasted_iota(jnp.int32, (_W, _OW), 1)
        kwv = iw2 - 2 * ow2 + 1
        tiles = []
        for kh in range(_KH):
            t = jnp.zeros((_W, _OW), jnp.float32)
            for kw in range(_KW):
                t = jnp.where(kwv == kw, w_ref[0, 0, kh, kw], t)
            tiles.append(t)
        zt = jnp.zeros((_W, _OW), jnp.float32)
        bands = []
        for ihv in range(_H):
            pieces = [zt] * _OH
            for kh in range(_KH):
                t2 = ihv + 1 - kh
                if t2 >= 0 and t2 % 2 == 0 and t2 // 2 < _OH:
                    pieces[t2 // 2] = tiles[kh]
            bands.append(jnp.concatenate(pieces, axis=1))
        return jnp.concatenate(bands, axis=0)

    def edge_block():
        return jnp.zeros((_A_BLK_R, _A_BLK_C), jnp.float32)

    out_ref[...] = jax.lax.cond((i < _C) & (j < _OC), tap_block, edge_block)


def _build_a(conv_weight, conv_bias):
    grid = (pl.cdiv(_ROWS_A, _A_BLK_R), pl.cdiv(_COLS_A, _A_BLK_C))
    a = pl.pallas_call(
        _a_build_kernel,
        grid=grid,
        in_specs=[
            pl.BlockSpec(
                (1, 1, _KH, _KW),
                lambda i, j: (jnp.minimum(j, _OC - 1), jnp.minimum(i, _C - 1), 0, 0),
            ),
        ],
        out_specs=pl.BlockSpec((_A_BLK_R, _A_BLK_C), lambda i, j: (i, j)),
        out_shape=jax.ShapeDtypeStruct((_ROWS_A, _COLS_A), jnp.float32),
    )(conv_weight)
    bias_row = jnp.concatenate(
        [jnp.repeat(conv_bias, _OUT // _OC), jnp.ones((1,), jnp.float32)]
    )
    return a.at[_PREV, :].set(bias_row)


_NW = 32
_RPW = _PREV // _NW
_RPC = 8
_CPW = _RPW // _RPC
_CHUNK = _RPC * _COLS_A
_BUF_N = 32784


def _a_sc_body(w_hbm, a_hbm, w_v, buf0, buf1, sem0, sem1):
    wid = lax.axis_index("s") * 2 + lax.axis_index("c")
    pltpu.sync_copy(w_hbm, w_v)
    bufs = (buf0, buf1)
    sems = (sem0, sem1)
    lanes = lax.iota(jnp.int32, 16)
    oc_col = lanes * (_OH * _OW)
    w_base = lanes * (_C * _KH * _KW)
    zeros16 = jnp.zeros((16,), jnp.float32)

    for b in range(2):
        for t in range(_BUF_N // 16):
            bufs[b][pl.ds(t * 16, 16)] = zeros16

    row0 = wid * _RPW

    def scatter_taps(buf, p_base, zero_mode):
        for rr in range(_RPC):
            p = p_base + rr
            cch = p // (_H * _W)
            rem = p % (_H * _W)
            ih = rem // _W
            iw = rem % _W
            rbase = rr * _COLS_A
            for kh in range(_KH):
                th = ih + 1 - kh
                oh = jnp.clip(th // 2, 0, _OH - 1)
                vh = (th >= 0) & (th < _H) & (th % 2 == 0)
                for kw in range(_KW):
                    tw = iw + 1 - kw
                    ow = jnp.clip(tw // 2, 0, _OW - 1)
                    vw = (tw >= 0) & (tw < _W) & (tw % 2 == 0)
                    mask = jnp.broadcast_to(vh & vw, (16,))
                    idx16 = oc_col + (rbase + oh * _OW + ow)
                    if zero_mode:
                        vals = zeros16
                    else:
                        vals = plsc.load_gather(
                            w_v, [w_base + cch * (_KH * _KW) + kh * _KW + kw])
                    plsc.store_scatter(buf, [idx16], vals, mask=mask)

    def loop_body(k2, carry):
        k = k2 * 2
        for b in range(2):
            kk = k + b

            @pl.when(kk >= 2)
            def _():
                pltpu.make_async_copy(
                    bufs[b].at[pl.ds(0, _CHUNK)],
                    a_hbm.at[pl.ds(0, _CHUNK)], sems[b]).wait()
                scatter_taps(bufs[b], row0 + (kk - 2) * _RPC, True)

            scatter_taps(bufs[b], row0 + kk * _RPC, False)
            pltpu.async_copy(
                bufs[b].at[pl.ds(0, _CHUNK)],
                a_hbm.at[pl.ds((row0 + kk * _RPC) * _COLS_A, _CHUNK)],
                sems[b])
        return carry

    lax.fori_loop(0, _CPW // 2, loop_body, 0)
    for b in range(2):
        pltpu.make_async_copy(
            bufs[b].at[pl.ds(0, _CHUNK)],
            a_hbm.at[pl.ds(0, _CHUNK)], sems[b]).wait()


def _build_a_sc(conv_weight, conv_bias):
    mesh = plsc.VectorSubcoreMesh(core_axis_name="c", subcore_axis_name="s")
    fn = functools.partial(
        pl.kernel,
        mesh=mesh,
        compiler_params=pltpu.CompilerParams(
            use_tc_tiling_on_sc=False, needs_layout_passes=False),
        out_type=jax.ShapeDtypeStruct((_ROWS_A * _COLS_A,), jnp.float32),
        scratch_types=[
            pltpu.VMEM((_OC * _C * _KH * _KW,), jnp.float32),
            pltpu.VMEM((_BUF_N,), jnp.float32),
            pltpu.VMEM((_BUF_N,), jnp.float32),
            pltpu.SemaphoreType.DMA,
            pltpu.SemaphoreType.DMA,
        ],
    )(_a_sc_body)
    a = fn(conv_weight.reshape(-1)).reshape(_ROWS_A, _COLS_A)
    bias_row = jnp.concatenate(
        [jnp.repeat(conv_bias, _OUT // _OC), jnp.ones((1,), jnp.float32)]
    )
    return a.at[_PREV, :].set(bias_row)


def _bounds_kernel(m_ref, a_ref, ce_ref, re_ref, low_ref, up_ref):
    b = jnp.dot(m_ref[...].astype(jnp.bfloat16),
                a_ref[...].astype(jnp.bfloat16),
                preferred_element_type=jnp.float32)
    t1 = jnp.dot(ce_ref[...], b, preferred_element_type=jnp.float32)
    t2 = jnp.dot(re_ref[...], jnp.abs(b), preferred_element_type=jnp.float32)
    low_ref[...] = t1 - t2
    up_ref[...] = t1 + t2


def _bounds(m, a, ce, re):
    n_tiles = pl.cdiv(_COLS_A, _N_TILE)
    low, up = pl.pallas_call(
        _bounds_kernel,
        grid=(n_tiles,),
        in_specs=[
            pl.BlockSpec((_D_IN, _ROWS_A), lambda n: (0, 0)),
            pl.BlockSpec((_ROWS_A, _N_TILE), lambda n: (0, n)),
            pl.BlockSpec((1, _D_IN), lambda n: (0, 0)),
            pl.BlockSpec((1, _D_IN), lambda n: (0, 0)),
        ],
        out_specs=[
            pl.BlockSpec((1, _N_TILE), lambda n: (0, n)),
            pl.BlockSpec((1, _N_TILE), lambda n: (0, n)),
        ],
        out_shape=[
            jax.ShapeDtypeStruct((1, n_tiles * _N_TILE), jnp.float32),
            jax.ShapeDtypeStruct((1, n_tiles * _N_TILE), jnp.float32),
        ],
    )(m, a, ce, re)
    return low, up


@jax.jit
def kernel(concrete_lower, concrete_upper, abstract_lower, abstract_upper,
           conv_weight, conv_bias, M, box_lower, box_upper):
    a = _build_a(conv_weight, conv_bias)
    c = (box_lower + box_upper) * 0.5
    r = (box_upper - box_lower) * 0.5
    ce = jnp.concatenate([c, jnp.ones((1,), jnp.float32)])[None, :]
    re = jnp.concatenate([r, jnp.zeros((1,), jnp.float32)])[None, :]
    low, up = _bounds(M, a, ce, re)
    out_dim = (_OC, _OH, _OW)
    lower_out = low[0, :_OUT].reshape(out_dim)
    upper_out = up[0, :_OUT].reshape(out_dim)
    return (lower_out, upper_out, a, a)


# speedup vs baseline: 1.1557x; 1.1180x over previous
import functools

import jax
import jax.numpy as jnp
from jax import lax
from jax.experimental import pallas as pl
from jax.experimental.pallas import tpu as pltpu
from jax.experimental.pallas import tpu_sc as plsc

_C, _H, _W = 3, 32, 32
_OC, _OH, _OW = 16, 16, 16
_KH, _KW = 4, 4
_PREV = _C * _H * _W
_OUT = _OC * _OH * _OW
_ROWS_A = _PREV + 1
_COLS_A = _OUT + 1
_D_IN = 1025

_A_BLK_R, _A_BLK_C = 1024, 512
_OC_PER_BLK = _A_BLK_C // (_OH * _OW)
_N_TILE = 512


def _a_build_kernel(w_ref, out_ref):
    i = pl.program_id(0)
    j = pl.program_id(1)

    def tap_block():
        iw2 = jax.lax.broadcasted_iota(jnp.int32, (_W, _OW), 0)
        ow2 = jax.lax.broadcasted_iota(jnp.int32, (_W, _OW), 1)
        kwv = iw2 - 2 * ow2 + 1
        tiles = []
        for u in range(_OC_PER_BLK):
            for kh in range(_KH):
                t = jnp.zeros((_W, _OW), jnp.float32)
                for kw in range(_KW):
                    t = jnp.where(kwv == kw, w_ref[u, 0, kh, kw], t)
                tiles.append(t)
        zt = jnp.zeros((_W, _OW), jnp.float32)
        bands = []
        for ihv in range(_H):
            pieces = [zt] * (_OH * _OC_PER_BLK)
            for u in range(_OC_PER_BLK):
                for kh in range(_KH):
                    t2 = ihv + 1 - kh
                    if t2 >= 0 and t2 % 2 == 0 and t2 // 2 < _OH:
                        pieces[u * _OH + t2 // 2] = tiles[u * _KH + kh]
            bands.append(jnp.concatenate(pieces, axis=1))
        return jnp.concatenate(bands, axis=0)

    def edge_block():
        return jnp.zeros((_A_BLK_R, _A_BLK_C), jnp.float32)

    out_ref[...] = jax.lax.cond(
        (i < _C) & (j < _OC // _OC_PER_BLK), tap_block, edge_block)


def _build_a(conv_weight, conv_bias):
    grid = (pl.cdiv(_ROWS_A, _A_BLK_R), pl.cdiv(_COLS_A, _A_BLK_C))
    a = pl.pallas_call(
        _a_build_kernel,
        grid=grid,
        in_specs=[
            pl.BlockSpec(
                (_OC_PER_BLK, 1, _KH, _KW),
                lambda i, j: (jnp.minimum(j, _OC // _OC_PER_BLK - 1),
                              jnp.minimum(i, _C - 1), 0, 0),
            ),
        ],
        out_specs=pl.BlockSpec((_A_BLK_R, _A_BLK_C), lambda i, j: (i, j)),
        out_shape=jax.ShapeDtypeStruct((_ROWS_A, _COLS_A), jnp.float32),
    )(conv_weight)
    bias_row = jnp.concatenate(
        [jnp.repeat(conv_bias, _OUT // _OC), jnp.ones((1,), jnp.float32)]
    )
    return a.at[_PREV, :].set(bias_row)


_NW = 32
_RPW = _PREV // _NW
_RPC = 8
_CPW = _RPW // _RPC
_CHUNK = _RPC * _COLS_A
_BUF_N = 32784


def _a_sc_body(w_hbm, a_hbm, w_v, buf0, buf1, sem0, sem1):
    wid = lax.axis_index("s") * 2 + lax.axis_index("c")
    pltpu.sync_copy(w_hbm, w_v)
    bufs = (buf0, buf1)
    sems = (sem0, sem1)
    lanes = lax.iota(jnp.int32, 16)
    oc_col = lanes * (_OH * _OW)
    w_base = lanes * (_C * _KH * _KW)
    zeros16 = jnp.zeros((16,), jnp.float32)

    for b in range(2):
        for t in range(_BUF_N // 16):
            bufs[b][pl.ds(t * 16, 16)] = zeros16

    row0 = wid * _RPW

    def scatter_taps(buf, p_base, zero_mode):
        for rr in range(_RPC):
            p = p_base + rr
            cch = p // (_H * _W)
            rem = p % (_H * _W)
            ih = rem // _W
            iw = rem % _W
            rbase = rr * _COLS_A
            for kh in range(_KH):
                th = ih + 1 - kh
                oh = jnp.clip(th // 2, 0, _OH - 1)
                vh = (th >= 0) & (th < _H) & (th % 2 == 0)
                for kw in range(_KW):
                    tw = iw + 1 - kw
                    ow = jnp.clip(tw // 2, 0, _OW - 1)
                    vw = (tw >= 0) & (tw < _W) & (tw % 2 == 0)
                    mask = jnp.broadcast_to(vh & vw, (16,))
                    idx16 = oc_col + (rbase + oh * _OW + ow)
                    if zero_mode:
                        vals = zeros16
                    else:
                        vals = plsc.load_gather(
                            w_v, [w_base + cch * (_KH * _KW) + kh * _KW + kw])
                    plsc.store_scatter(buf, [idx16], vals, mask=mask)

    def loop_body(k2, carry):
        k = k2 * 2
        for b in range(2):
            kk = k + b

            @pl.when(kk >= 2)
            def _():
                pltpu.make_async_copy(
                    bufs[b].at[pl.ds(0, _CHUNK)],
                    a_hbm.at[pl.ds(0, _CHUNK)], sems[b]).wait()
                scatter_taps(bufs[b], row0 + (kk - 2) * _RPC, True)

            scatter_taps(bufs[b], row0 + kk * _RPC, False)
            pltpu.async_copy(
                bufs[b].at[pl.ds(0, _CHUNK)],
                a_hbm.at[pl.ds((row0 + kk * _RPC) * _COLS_A, _CHUNK)],
                sems[b])
        return carry

    lax.fori_loop(0, _CPW // 2, loop_body, 0)
    for b in range(2):
        pltpu.make_async_copy(
            bufs[b].at[pl.ds(0, _CHUNK)],
            a_hbm.at[pl.ds(0, _CHUNK)], sems[b]).wait()


def _build_a_sc(conv_weight, conv_bias):
    mesh = plsc.VectorSubcoreMesh(core_axis_name="c", subcore_axis_name="s")
    fn = functools.partial(
        pl.kernel,
        mesh=mesh,
        compiler_params=pltpu.CompilerParams(
            use_tc_tiling_on_sc=False, needs_layout_passes=False),
        out_type=jax.ShapeDtypeStruct((_ROWS_A * _COLS_A,), jnp.float32),
        scratch_types=[
            pltpu.VMEM((_OC * _C * _KH * _KW,), jnp.float32),
            pltpu.VMEM((_BUF_N,), jnp.float32),
            pltpu.VMEM((_BUF_N,), jnp.float32),
            pltpu.SemaphoreType.DMA,
            pltpu.SemaphoreType.DMA,
        ],
    )(_a_sc_body)
    a = fn(conv_weight.reshape(-1)).reshape(_ROWS_A, _COLS_A)
    bias_row = jnp.concatenate(
        [jnp.repeat(conv_bias, _OUT // _OC), jnp.ones((1,), jnp.float32)]
    )
    return a.at[_PREV, :].set(bias_row)


def _bounds_kernel(m_ref, a_ref, ce_ref, re_ref, low_ref, up_ref):
    b = jnp.dot(m_ref[...], a_ref[...], preferred_element_type=jnp.float32)
    t1 = jnp.dot(ce_ref[...], b, preferred_element_type=jnp.float32)
    t2 = jnp.dot(re_ref[...], jnp.abs(b), preferred_element_type=jnp.float32)
    low_ref[...] = t1 - t2
    up_ref[...] = t1 + t2


def _bounds(m, a, ce, re):
    n_tiles = pl.cdiv(_COLS_A, _N_TILE)
    low, up = pl.pallas_call(
        _bounds_kernel,
        grid=(n_tiles,),
        in_specs=[
            pl.BlockSpec((_D_IN, _ROWS_A), lambda n: (0, 0)),
            pl.BlockSpec((_ROWS_A, _N_TILE), lambda n: (0, n)),
            pl.BlockSpec((1, _D_IN), lambda n: (0, 0)),
            pl.BlockSpec((1, _D_IN), lambda n: (0, 0)),
        ],
        out_specs=[
            pl.BlockSpec((1, _N_TILE), lambda n: (0, n)),
            pl.BlockSpec((1, _N_TILE), lambda n: (0, n)),
        ],
        out_shape=[
            jax.ShapeDtypeStruct((1, n_tiles * _N_TILE), jnp.float32),
            jax.ShapeDtypeStruct((1, n_tiles * _N_TILE), jnp.float32),
        ],
    )(m, a, ce, re)
    return low, up


@jax.jit
def kernel(concrete_lower, concrete_upper, abstract_lower, abstract_upper,
           conv_weight, conv_bias, M, box_lower, box_upper):
    a = _build_a(conv_weight, conv_bias)
    c = (box_lower + box_upper) * 0.5
    r = (box_upper - box_lower) * 0.5
    ce = jnp.concatenate([c, jnp.ones((1,), jnp.float32)])[None, :]
    re = jnp.concatenate([r, jnp.zeros((1,), jnp.float32)])[None, :]
    low, up = _bounds(M, a, ce, re)
    out_dim = (_OC, _OH, _OW)
    lower_out = low[0, :_OUT].reshape(out_dim)
    upper_out = up[0, :_OUT].reshape(out_dim)
    return (lower_out, upper_out, a, a)


# speedup vs baseline: 1.2039x; 1.0417x over previous
import functools

import jax
import jax.numpy as jnp
from jax import lax
from jax.experimental import pallas as pl
from jax.experimental.pallas import tpu as pltpu
from jax.experimental.pallas import tpu_sc as plsc

_C, _H, _W = 3, 32, 32
_OC, _OH, _OW = 16, 16, 16
_KH, _KW = 4, 4
_PREV = _C * _H * _W
_OUT = _OC * _OH * _OW
_ROWS_A = _PREV + 1
_COLS_A = _OUT + 1
_D_IN = 1025

_A_BLK_R, _A_BLK_C = 1024, 1024
_OC_PER_BLK = _A_BLK_C // (_OH * _OW)
_N_TILE = 512


def _a_build_kernel(w_ref, out_ref):
    i = pl.program_id(0)
    j = pl.program_id(1)

    def tap_block():
        iw2 = jax.lax.broadcasted_iota(jnp.int32, (_W, _OW), 0)
        ow2 = jax.lax.broadcasted_iota(jnp.int32, (_W, _OW), 1)
        kwv = iw2 - 2 * ow2 + 1
        tiles = []
        for u in range(_OC_PER_BLK):
            for kh in range(_KH):
                t = jnp.zeros((_W, _OW), jnp.float32)
                for kw in range(_KW):
                    t = jnp.where(kwv == kw, w_ref[u, 0, kh, kw], t)
                tiles.append(t)
        zt = jnp.zeros((_W, _OW), jnp.float32)
        bands = []
        for ihv in range(_H):
            pieces = [zt] * (_OH * _OC_PER_BLK)
            for u in range(_OC_PER_BLK):
                for kh in range(_KH):
                    t2 = ihv + 1 - kh
                    if t2 >= 0 and t2 % 2 == 0 and t2 // 2 < _OH:
                        pieces[u * _OH + t2 // 2] = tiles[u * _KH + kh]
            bands.append(jnp.concatenate(pieces, axis=1))
        return jnp.concatenate(bands, axis=0)

    def edge_block():
        return jnp.zeros((_A_BLK_R, _A_BLK_C), jnp.float32)

    out_ref[...] = jax.lax.cond(
        (i < _C) & (j < _OC // _OC_PER_BLK), tap_block, edge_block)


def _build_a(conv_weight, conv_bias):
    grid = (pl.cdiv(_ROWS_A, _A_BLK_R), pl.cdiv(_COLS_A, _A_BLK_C))
    a = pl.pallas_call(
        _a_build_kernel,
        grid=grid,
        in_specs=[
            pl.BlockSpec(
                (_OC_PER_BLK, 1, _KH, _KW),
                lambda i, j: (jnp.minimum(j, _OC // _OC_PER_BLK - 1),
                              jnp.minimum(i, _C - 1), 0, 0),
            ),
        ],
        out_specs=pl.BlockSpec((_A_BLK_R, _A_BLK_C), lambda i, j: (i, j)),
        out_shape=jax.ShapeDtypeStruct((_ROWS_A, _COLS_A), jnp.float32),
    )(conv_weight)
    bias_row = jnp.concatenate(
        [jnp.repeat(conv_bias, _OUT // _OC), jnp.ones((1,), jnp.float32)]
    )
    return a.at[_PREV, :].set(bias_row)


_NW = 32
_RPW = _PREV // _NW
_RPC = 8
_CPW = _RPW // _RPC
_CHUNK = _RPC * _COLS_A
_BUF_N = 32784


def _a_sc_body(w_hbm, a_hbm, w_v, buf0, buf1, sem0, sem1):
    wid = lax.axis_index("s") * 2 + lax.axis_index("c")
    pltpu.sync_copy(w_hbm, w_v)
    bufs = (buf0, buf1)
    sems = (sem0, sem1)
    lanes = lax.iota(jnp.int32, 16)
    oc_col = lanes * (_OH * _OW)
    w_base = lanes * (_C * _KH * _KW)
    zeros16 = jnp.zeros((16,), jnp.float32)

    for b in range(2):
        for t in range(_BUF_N // 16):
            bufs[b][pl.ds(t * 16, 16)] = zeros16

    row0 = wid * _RPW

    def scatter_taps(buf, p_base, zero_mode):
        for rr in range(_RPC):
            p = p_base + rr
            cch = p // (_H * _W)
            rem = p % (_H * _W)
            ih = rem // _W
            iw = rem % _W
            rbase = rr * _COLS_A
            for kh in range(_KH):
                th = ih + 1 - kh
                oh = jnp.clip(th // 2, 0, _OH - 1)
                vh = (th >= 0) & (th < _H) & (th % 2 == 0)
                for kw in range(_KW):
                    tw = iw + 1 - kw
                    ow = jnp.clip(tw // 2, 0, _OW - 1)
                    vw = (tw >= 0) & (tw < _W) & (tw % 2 == 0)
                    mask = jnp.broadcast_to(vh & vw, (16,))
                    idx16 = oc_col + (rbase + oh * _OW + ow)
                    if zero_mode:
                        vals = zeros16
                    else:
                        vals = plsc.load_gather(
                            w_v, [w_base + cch * (_KH * _KW) + kh * _KW + kw])
                    plsc.store_scatter(buf, [idx16], vals, mask=mask)

    def loop_body(k2, carry):
        k = k2 * 2
        for b in range(2):
            kk = k + b

            @pl.when(kk >= 2)
            def _():
                pltpu.make_async_copy(
                    bufs[b].at[pl.ds(0, _CHUNK)],
                    a_hbm.at[pl.ds(0, _CHUNK)], sems[b]).wait()
                scatter_taps(bufs[b], row0 + (kk - 2) * _RPC, True)

            scatter_taps(bufs[b], row0 + kk * _RPC, False)
            pltpu.async_copy(
                bufs[b].at[pl.ds(0, _CHUNK)],
                a_hbm.at[pl.ds((row0 + kk * _RPC) * _COLS_A, _CHUNK)],
                sems[b])
        return carry

    lax.fori_loop(0, _CPW // 2, loop_body, 0)
    for b in range(2):
        pltpu.make_async_copy(
            bufs[b].at[pl.ds(0, _CHUNK)],
            a_hbm.at[pl.ds(0, _CHUNK)], sems[b]).wait()


def _build_a_sc(conv_weight, conv_bias):
    mesh = plsc.VectorSubcoreMesh(core_axis_name="c", subcore_axis_name="s")
    fn = functools.partial(
        pl.kernel,
        mesh=mesh,
        compiler_params=pltpu.CompilerParams(
            use_tc_tiling_on_sc=False, needs_layout_passes=False),
        out_type=jax.ShapeDtypeStruct((_ROWS_A * _COLS_A,), jnp.float32),
        scratch_types=[
            pltpu.VMEM((_OC * _C * _KH * _KW,), jnp.float32),
            pltpu.VMEM((_BUF_N,), jnp.float32),
            pltpu.VMEM((_BUF_N,), jnp.float32),
            pltpu.SemaphoreType.DMA,
            pltpu.SemaphoreType.DMA,
        ],
    )(_a_sc_body)
    a = fn(conv_weight.reshape(-1)).reshape(_ROWS_A, _COLS_A)
    bias_row = jnp.concatenate(
        [jnp.repeat(conv_bias, _OUT // _OC), jnp.ones((1,), jnp.float32)]
    )
    return a.at[_PREV, :].set(bias_row)


def _bounds_kernel(m_ref, a_ref, ce_ref, re_ref, low_ref, up_ref):
    b = jnp.dot(m_ref[...], a_ref[...], preferred_element_type=jnp.float32)
    t1 = jnp.dot(ce_ref[...], b, preferred_element_type=jnp.float32)
    t2 = jnp.dot(re_ref[...], jnp.abs(b), preferred_element_type=jnp.float32)
    low_ref[...] = t1 - t2
    up_ref[...] = t1 + t2


def _bounds(m, a, ce, re):
    n_tiles = pl.cdiv(_COLS_A, _N_TILE)
    low, up = pl.pallas_call(
        _bounds_kernel,
        grid=(n_tiles,),
        in_specs=[
            pl.BlockSpec((_D_IN, _ROWS_A), lambda n: (0, 0)),
            pl.BlockSpec((_ROWS_A, _N_TILE), lambda n: (0, n)),
            pl.BlockSpec((1, _D_IN), lambda n: (0, 0)),
            pl.BlockSpec((1, _D_IN), lambda n: (0, 0)),
        ],
        out_specs=[
            pl.BlockSpec((1, _N_TILE), lambda n: (0, n)),
            pl.BlockSpec((1, _N_TILE), lambda n: (0, n)),
        ],
        out_shape=[
            jax.ShapeDtypeStruct((1, n_tiles * _N_TILE), jnp.float32),
            jax.ShapeDtypeStruct((1, n_tiles * _N_TILE), jnp.float32),
        ],
    )(m, a, ce, re)
    return low, up


@jax.jit
def kernel(concrete_lower, concrete_upper, abstract_lower, abstract_upper,
           conv_weight, conv_bias, M, box_lower, box_upper):
    a = _build_a(conv_weight, conv_bias)
    c = (box_lower + box_upper) * 0.5
    r = (box_upper - box_lower) * 0.5
    ce = jnp.concatenate([c, jnp.ones((1,), jnp.float32)])[None, :]
    re = jnp.concatenate([r, jnp.zeros((1,), jnp.float32)])[None, :]
    low, up = _bounds(M, a, ce, re)
    out_dim = (_OC, _OH, _OW)
    lower_out = low[0, :_OUT].reshape(out_dim)
    upper_out = up[0, :_OUT].reshape(out_dim)
    return (lower_out, upper_out, a, a)
